# baseline (speedup 1.0000x reference)
import functools

import jax
import jax.numpy as jnp
from jax.experimental import pallas as pl
from jax.experimental.pallas import tpu as pltpu

_R = 256
_CB = 2048


def _margin_kernel(k_ref, pred_ref, out_ref, max_acc, pk_acc, *, C):
    j = pl.program_id(1)
    nj = pl.num_programs(1)

    @pl.when(j == 0)
    def _init():
        max_acc[...] = jnp.full_like(max_acc, -jnp.inf)
        pk_acc[...] = jnp.zeros_like(pk_acc)

    x = pred_ref[...]
    col = j * _CB + jax.lax.broadcasted_iota(jnp.int32, x.shape, 1)
    krow = k_ref[...]
    is_k = col == krow
    valid = col < C
    tile_max = jnp.where(valid & (~is_k), x, -jnp.inf).max(axis=1, keepdims=True)
    max_acc[...] = jnp.maximum(max_acc[...], tile_max)
    pk_acc[...] += jnp.where(is_k, x, 0.0).sum(axis=1, keepdims=True)

    @pl.when(j == nj - 1)
    def _finish():
        out_ref[...] = max_acc[...] - pk_acc[...]


def kernel(prediction, k):
    B, C = prediction.shape
    k2 = k.astype(jnp.int32).reshape(B, 1)
    ncb = pl.cdiv(C, _CB)
    out = pl.pallas_call(
        functools.partial(_margin_kernel, C=C),
        grid=(B // _R, ncb),
        in_specs=[
            pl.BlockSpec((_R, 1), lambda i, j: (i, 0)),
            pl.BlockSpec((_R, _CB), lambda i, j: (i, j)),
        ],
        out_specs=pl.BlockSpec((_R, 1), lambda i, j: (i, 0)),
        out_shape=jax.ShapeDtypeStruct((B, 1), jnp.float32),
        scratch_shapes=[
            pltpu.VMEM((_R, 1), jnp.float32),
            pltpu.VMEM((_R, 1), jnp.float32),
        ],
        compiler_params=pltpu.CompilerParams(
            dimension_semantics=("parallel", "arbitrary"),
        ),
    )(k2, prediction)
    return out.reshape(B)


# speedup vs baseline: 1.1806x; 1.1806x over previous
import functools

import jax
import jax.numpy as jnp
from jax.experimental import pallas as pl
from jax.experimental.pallas import tpu as pltpu

_R = 8


def _margin_kernel(k_ref, pred_ref, out_ref, *, C):
    i = pl.program_id(0)
    C_al = (C // 128) * 128

    lane = jax.lax.broadcasted_iota(jnp.int32, (1, 128), 1)
    pks = []
    for r in range(_R):
        c = k_ref[i * _R + r]
        c0 = (c // 128) * 128
        chunk = pred_ref[pl.ds(r, 1), pl.ds(c0, 128)]
        is_l = lane == (c - c0)
        pks.append(jnp.where(is_l, chunk, -jnp.inf).max(axis=1, keepdims=True))
        pred_ref[pl.ds(r, 1), pl.ds(c0, 128)] = jnp.where(is_l, -jnp.inf, chunk)

    main = pred_ref[:, :C_al]
    m = jnp.max(main, axis=1)
    tail = pred_ref[:, C_al:]
    tmask = jax.lax.broadcasted_iota(jnp.int32, tail.shape, 1) < (C - C_al)
    m = jnp.maximum(m, jnp.where(tmask, tail, -jnp.inf).max(axis=1))

    pk = jnp.concatenate(pks, axis=0)
    out_ref[...] = m[:, None] - pk


def kernel(prediction, k):
    B, C = prediction.shape
    k2 = k.astype(jnp.int32)
    C_pad = ((C + 127) // 128) * 128
    out = pl.pallas_call(
        functools.partial(_margin_kernel, C=C),
        grid=(B // _R,),
        in_specs=[
            pl.BlockSpec(memory_space=pltpu.SMEM),
            pl.BlockSpec((_R, C_pad), lambda i: (i, 0)),
        ],
        out_specs=pl.BlockSpec((_R, 1), lambda i: (i, 0)),
        out_shape=jax.ShapeDtypeStruct((B, 1), jnp.float32),
        compiler_params=pltpu.CompilerParams(
            dimension_semantics=("arbitrary",),
        ),
    )(k2, prediction)
    return out.reshape(B)


# speedup vs baseline: 1.2582x; 1.0657x over previous
import functools

import jax
import jax.numpy as jnp
from jax.experimental import pallas as pl
from jax.experimental.pallas import tpu as pltpu

_R = 32


def _margin_kernel(k_ref, pred_ref, out_ref, *, C):
    i = pl.program_id(0)
    C_al = (C // 128) * 128

    lane = jax.lax.broadcasted_iota(jnp.int32, (1, 128), 1)
    pks = []
    for r in range(_R):
        c = k_ref[i * _R + r]
        c0 = (c // 128) * 128
        chunk = pred_ref[pl.ds(r, 1), pl.ds(c0, 128)]
        is_l = lane == (c - c0)
        pks.append(jnp.where(is_l, chunk, -jnp.inf).max(axis=1, keepdims=True))
        pred_ref[pl.ds(r, 1), pl.ds(c0, 128)] = jnp.where(is_l, -jnp.inf, chunk)

    main = pred_ref[:, :C_al]
    m = jnp.max(main, axis=1)
    tail = pred_ref[:, C_al:]
    tmask = jax.lax.broadcasted_iota(jnp.int32, tail.shape, 1) < (C - C_al)
    m = jnp.maximum(m, jnp.where(tmask, tail, -jnp.inf).max(axis=1))

    pk = jnp.concatenate(pks, axis=0)
    out_ref[...] = m[:, None] - pk


def kernel(prediction, k):
    B, C = prediction.shape
    k2 = k.astype(jnp.int32)
    C_pad = ((C + 127) // 128) * 128
    out = pl.pallas_call(
        functools.partial(_margin_kernel, C=C),
        grid=(B // _R,),
        in_specs=[
            pl.BlockSpec(memory_space=pltpu.SMEM),
            pl.BlockSpec((_R, C_pad), lambda i: (i, 0)),
        ],
        out_specs=pl.BlockSpec((_R, 1), lambda i: (i, 0)),
        out_shape=jax.ShapeDtypeStruct((B, 1), jnp.float32),
        compiler_params=pltpu.CompilerParams(
            dimension_semantics=("arbitrary",),
        ),
    )(k2, prediction)
    return out.reshape(B)
